# baseline (speedup 1.0000x reference)
import functools

import jax
import jax.numpy as jnp
from jax import lax
from jax.experimental import pallas as pl
from jax.experimental.pallas import tpu as pltpu
from jax.experimental.pallas import tpu_sc as plsc

N = 10000
E = 320000
LATENT = 128
DOUT = 64

NC = 2
NS = 16
NW = NC * NS
BATCH = 128
CHB = 16
NCH = 5
NB = NCH * CHB
EPW = NB * BATCH
EPAD = NW * EPW - E
ACCN = N + 16
RPS = 624
RTAIL = N - RPS * NS


def _make_spmm(d):
    mesh = plsc.VectorSubcoreMesh(core_axis_name="c", subcore_axis_name="s")

    @functools.partial(
        pl.kernel,
        out_type=jax.ShapeDtypeStruct((NC, N, d), jnp.float32),
        mesh=mesh,
        scratch_types=[
            pltpu.VMEM((2, CHB, BATCH), jnp.int32),
            pltpu.VMEM((2, CHB, BATCH), jnp.int32),
            pltpu.VMEM((2, BATCH, d), jnp.float32),
            pltpu.VMEM_SHARED((ACCN, d), jnp.float32),
            pltpu.SemaphoreType.DMA,
            pltpu.SemaphoreType.DMA,
            pltpu.SemaphoreType.DMA,
            pltpu.SemaphoreType.DMA,
        ],
    )
    def spmm(support, src, dst, zeros, out,
             src_v, dst_v, rows_v, acc, g0, g1, is_, id_):
        cid = lax.axis_index("c")
        sid = lax.axis_index("s")
        wid = sid * NC + cid

        def ichunk(c, b):
            pltpu.async_copy(src.at[wid, pl.ds(c * CHB, CHB)], src_v.at[b], is_)
            pltpu.async_copy(dst.at[wid, pl.ds(c * CHB, CHB)], dst_v.at[b], id_)

        def ichunk_wait(b):
            pltpu.make_async_copy(src.at[0, pl.ds(0, CHB)], src_v.at[b], is_).wait()
            pltpu.make_async_copy(dst.at[0, pl.ds(0, CHB)], dst_v.at[b], id_).wait()

        def gissue(b, r, s, sem):
            pltpu.async_copy(support.at[src_v.at[b, r]], rows_v.at[s], sem)

        def gwait(b, r, s, sem):
            pltpu.make_async_copy(
                support.at[src_v.at[b, r]], rows_v.at[s], sem).wait()

        ichunk(0, 0)
        off = pl.multiple_of(sid * RPS, 8)
        pltpu.sync_copy(zeros.at[pl.ds(off, RPS)], acc.at[pl.ds(off, RPS)])

        @pl.when(sid == NS - 1)
        def _():
            pltpu.sync_copy(zeros.at[pl.ds(RPS * NS, RTAIL)],
                            acc.at[pl.ds(RPS * NS, RTAIL)])

        plsc.subcore_barrier()

        ichunk_wait(0)
        ichunk(1, 1)
        gissue(0, 0, 0, g0)

        for c in range(NCH):
            cb = c % 2
            nb = (c + 1) % 2

            def pair(kk, carry, cb=cb):
                a = 2 * kk
                b = a + 1
                gwait(cb, a, 0, g0)
                gissue(cb, b, 1, g1)
                pltpu.sync_copy(rows_v.at[0], acc.at[dst_v.at[cb, a]], add=True)
                gwait(cb, b, 1, g1)

                @pl.when(b + 1 < CHB)
                def _():
                    gissue(cb, b + 1, 0, g0)

                pltpu.sync_copy(rows_v.at[1], acc.at[dst_v.at[cb, b]], add=True)
                return carry

            lax.fori_loop(0, CHB // 2, pair, 0)
            if c + 1 < NCH:
                ichunk_wait(nb)
                if c + 2 < NCH:
                    ichunk(c + 2, cb)
                gissue(nb, 0, 0, g0)

        plsc.subcore_barrier()
        pltpu.sync_copy(acc.at[pl.ds(off, RPS)], out.at[cid, pl.ds(off, RPS)])

        @pl.when(sid == NS - 1)
        def _():
            pltpu.sync_copy(acc.at[pl.ds(RPS * NS, RTAIL)],
                            out.at[cid, pl.ds(RPS * NS, RTAIL)])

    return spmm


_spmm128 = _make_spmm(LATENT)

BM = 1000
BDI = 1000
BDJ = 1280


def _tanh_mm(x, w):
    din, dout = w.shape

    def body(x_ref, w_ref, o_ref):
        o_ref[...] = jnp.tanh(
            jnp.dot(x_ref[...], w_ref[...], preferred_element_type=jnp.float32))

    return pl.pallas_call(
        body,
        grid=(N // BM,),
        in_specs=[pl.BlockSpec((BM, din), lambda i: (i, 0)),
                  pl.BlockSpec((din, dout), lambda i: (0, 0))],
        out_specs=pl.BlockSpec((BM, dout), lambda i: (i, 0)),
        out_shape=jax.ShapeDtypeStruct((N, dout), jnp.float32),
    )(x, w)


def _tanh_mm_partials(p, w):
    din, dout = w.shape

    def body(p_ref, w_ref, o_ref):
        x = p_ref[0] + p_ref[1]
        o_ref[...] = jnp.tanh(
            jnp.dot(x, w_ref[...], preferred_element_type=jnp.float32))

    return pl.pallas_call(
        body,
        grid=(N // BM,),
        in_specs=[pl.BlockSpec((2, BM, din), lambda i: (0, i, 0)),
                  pl.BlockSpec((din, dout), lambda i: (0, 0))],
        out_specs=pl.BlockSpec((BM, dout), lambda i: (i, 0)),
        out_shape=jax.ShapeDtypeStruct((N, dout), jnp.float32),
    )(p, w)


def _decode(p):
    def body(a_ref, b_ref, x_ref, adj_ref):
        xi = a_ref[0] + a_ref[1]
        xj = b_ref[0] + b_ref[1]
        x_ref[...] = xi[:, :DOUT]
        logits = lax.dot_general(xi, xj, (((1,), (1,)), ((), ())),
                                 preferred_element_type=jnp.float32)
        adj_ref[...] = jax.nn.sigmoid(logits)

    return pl.pallas_call(
        body,
        grid=(N // BDI, (N + BDJ - 1) // BDJ),
        in_specs=[pl.BlockSpec((2, BDI, LATENT), lambda i, j: (0, i, 0)),
                  pl.BlockSpec((2, BDJ, LATENT), lambda i, j: (0, j, 0))],
        out_specs=[pl.BlockSpec((BDI, DOUT), lambda i, j: (i, 0)),
                   pl.BlockSpec((BDI, BDJ), lambda i, j: (i, j))],
        out_shape=[jax.ShapeDtypeStruct((N, DOUT), jnp.float32),
                   jax.ShapeDtypeStruct((N, N), jnp.float32)],
    )(p, p)


def kernel(z_x, adj_edge_index, W4, W5, W6):
    dst = jnp.concatenate(
        [adj_edge_index[0], jnp.full((EPAD,), N, jnp.int32)]).reshape(
            NW, NB, BATCH)
    src = jnp.concatenate(
        [adj_edge_index[1], jnp.zeros((EPAD,), jnp.int32)]).reshape(
            NW, NB, BATCH)
    z128 = jnp.zeros((N, LATENT), jnp.float32)
    w6p = jnp.pad(W6, ((0, 0), (0, LATENT - DOUT)))

    s = _tanh_mm(z_x, W4)
    p = _spmm128(s, src, dst, z128)
    s = _tanh_mm_partials(p, W5)
    p = _spmm128(s, src, dst, z128)
    s = _tanh_mm_partials(p, w6p)
    p = _spmm128(s, src, dst, z128)
    x_hat, adj_hat = _decode(p)
    return (x_hat, adj_hat)


# speedup vs baseline: 1.1240x; 1.1240x over previous
import functools

import jax
import jax.numpy as jnp
from jax import lax
from jax.experimental import pallas as pl
from jax.experimental.pallas import tpu as pltpu
from jax.experimental.pallas import tpu_sc as plsc

N = 10000
E = 320000
LATENT = 128
DOUT = 64

NC = 2
NS = 16
NW = NC * NS
BATCH = 64
NB = 160
CHB = 8
NCH = NB // CHB
EPW = NB * BATCH
EPAD = NW * EPW - E
ACCN = N + 1
RPS = 624
RTAIL = N - RPS * NS
def _make_spmm(d, nbuf):
    assert CHB % nbuf == 0
    g = nbuf - 1
    mesh = plsc.VectorSubcoreMesh(core_axis_name="c", subcore_axis_name="s")

    @functools.partial(
        pl.kernel,
        out_type=jax.ShapeDtypeStruct((NC, N, d), jnp.float32),
        mesh=mesh,
        scratch_types=[
            pltpu.VMEM((2, CHB, BATCH), jnp.int32),
            pltpu.VMEM((2, CHB, BATCH), jnp.int32),
            pltpu.VMEM((nbuf, BATCH, d), jnp.float32),
            pltpu.VMEM_SHARED((ACCN, d), jnp.float32),
            pltpu.SemaphoreType.DMA,
        ] + [pltpu.SemaphoreType.DMA] * (2 * nbuf),
    )
    def spmm(support, src, dst, zeros, out,
             src_v, dst_v, rows_v, acc, isem, *sems):
        gs = sems[:nbuf]
        ss = sems[nbuf:]
        cid = lax.axis_index("c")
        sid = lax.axis_index("s")
        wid = sid * NC + cid

        def ichunk(c, b):
            pltpu.async_copy(src.at[wid, pl.ds(c * CHB, CHB)], src_v.at[b],
                             isem)
            pltpu.async_copy(dst.at[wid, pl.ds(c * CHB, CHB)], dst_v.at[b],
                             isem)

        def ichunk_wait():
            pltpu.make_async_copy(src.at[0, pl.ds(0, CHB)], src_v.at[0],
                                  isem).wait()
            pltpu.make_async_copy(dst.at[0, pl.ds(0, CHB)], dst_v.at[0],
                                  isem).wait()

        def gissue(buf, row, slot):
            pltpu.async_copy(support.at[src_v.at[buf, row]], rows_v.at[slot],
                             gs[slot])

        def gwait(slot):
            pltpu.make_async_copy(support.at[src_v.at[0, 0]],
                                  rows_v.at[slot], gs[slot]).wait()

        def sissue(buf, row, slot):
            pltpu.async_copy(rows_v.at[slot], acc.at[dst_v.at[buf, row]],
                             ss[slot], add=True)

        def swait(slot):
            pltpu.make_async_copy(rows_v.at[slot], acc.at[dst_v.at[0, 0]],
                                  ss[slot]).wait()

        ichunk(0, 0)
        off = pl.multiple_of(sid * RPS, 8)
        pltpu.sync_copy(zeros.at[pl.ds(off, RPS)], acc.at[pl.ds(off, RPS)])

        @pl.when(sid == NS - 1)
        def _():
            pltpu.sync_copy(zeros.at[pl.ds(RPS * NS, RTAIL)],
                            acc.at[pl.ds(RPS * NS, RTAIL)])

        plsc.subcore_barrier()
        ichunk_wait()

        for k in range(g):
            gissue(0, k, k)

        def chunk_steps(x, first, last):
            buf = lax.rem(x, 2) if not isinstance(x, int) else x % 2
            nbf = (lax.rem(x + 1, 2) if not isinstance(x, int)
                   else (x + 1) % 2)
            for k in range(CHB):
                slot = k % nbuf
                pslot = (k + g) % nbuf
                gwait(slot)
                sissue(buf, k, slot)
                if not (first and k == 0):
                    swait(pslot)
                if k == 0 and not last:
                    ichunk(x + 1, nbf)
                if k + g < CHB:
                    gissue(buf, k + g, pslot)
                elif not last:
                    gissue(nbf, k + g - CHB, pslot)
                if k == CHB - g - 1 and not last:
                    ichunk_wait()

        chunk_steps(0, first=True, last=False)

        def body(x, carry):
            chunk_steps(x, first=False, last=False)
            return carry

        lax.fori_loop(1, NCH - 1, body, 0)
        chunk_steps(NCH - 1, first=False, last=True)
        swait((NB - 1) % nbuf)

        plsc.subcore_barrier()
        pltpu.sync_copy(acc.at[pl.ds(off, RPS)], out.at[cid, pl.ds(off, RPS)])

        @pl.when(sid == NS - 1)
        def _():
            pltpu.sync_copy(acc.at[pl.ds(RPS * NS, RTAIL)],
                            out.at[cid, pl.ds(RPS * NS, RTAIL)])

    return spmm


_spmm128 = _make_spmm(LATENT, nbuf=4)

BM = 1000
BDI = 1000
BDJ = 1280


def _tanh_mm(x, w):
    din, dout = w.shape

    def body(x_ref, w_ref, o_ref):
        o_ref[...] = jnp.tanh(
            jnp.dot(x_ref[...], w_ref[...], preferred_element_type=jnp.float32))

    return pl.pallas_call(
        body,
        grid=(N // BM,),
        in_specs=[pl.BlockSpec((BM, din), lambda i: (i, 0)),
                  pl.BlockSpec((din, dout), lambda i: (0, 0))],
        out_specs=pl.BlockSpec((BM, dout), lambda i: (i, 0)),
        out_shape=jax.ShapeDtypeStruct((N, dout), jnp.float32),
    )(x, w)


def _tanh_mm_partials(p, w):
    din, dout = w.shape

    def body(p_ref, w_ref, o_ref):
        x = p_ref[0] + p_ref[1]
        o_ref[...] = jnp.tanh(
            jnp.dot(x, w_ref[...], preferred_element_type=jnp.float32))

    return pl.pallas_call(
        body,
        grid=(N // BM,),
        in_specs=[pl.BlockSpec((2, BM, din), lambda i: (0, i, 0)),
                  pl.BlockSpec((din, dout), lambda i: (0, 0))],
        out_specs=pl.BlockSpec((BM, dout), lambda i: (i, 0)),
        out_shape=jax.ShapeDtypeStruct((N, dout), jnp.float32),
    )(p, w)


def _decode(p):
    def body(a_ref, b_ref, x_ref, adj_ref):
        xi = a_ref[0] + a_ref[1]
        xj = b_ref[0] + b_ref[1]
        x_ref[...] = xi[:, :DOUT]
        logits = lax.dot_general(xi, xj, (((1,), (1,)), ((), ())),
                                 preferred_element_type=jnp.float32)
        adj_ref[...] = jax.nn.sigmoid(logits)

    return pl.pallas_call(
        body,
        grid=(N // BDI, (N + BDJ - 1) // BDJ),
        in_specs=[pl.BlockSpec((2, BDI, LATENT), lambda i, j: (0, i, 0)),
                  pl.BlockSpec((2, BDJ, LATENT), lambda i, j: (0, j, 0))],
        out_specs=[pl.BlockSpec((BDI, DOUT), lambda i, j: (i, 0)),
                   pl.BlockSpec((BDI, BDJ), lambda i, j: (i, j))],
        out_shape=[jax.ShapeDtypeStruct((N, DOUT), jnp.float32),
                   jax.ShapeDtypeStruct((N, N), jnp.float32)],
    )(p, p)


def kernel(z_x, adj_edge_index, W4, W5, W6):
    dst = jnp.concatenate(
        [adj_edge_index[0], jnp.full((EPAD,), N, jnp.int32)]).reshape(
            NW, NB, BATCH)
    src = jnp.concatenate(
        [adj_edge_index[1], jnp.zeros((EPAD,), jnp.int32)]).reshape(
            NW, NB, BATCH)
    z128 = jnp.zeros((N, LATENT), jnp.float32)
    w6p = jnp.pad(W6, ((0, 0), (0, LATENT - DOUT)))

    s = _tanh_mm(z_x, W4)
    p = _spmm128(s, src, dst, z128)
    s = _tanh_mm_partials(p, W5)
    p = _spmm128(s, src, dst, z128)
    s = _tanh_mm_partials(p, w6p)
    p = _spmm128(s, src, dst, z128)
    x_hat, adj_hat = _decode(p)
    return (x_hat, adj_hat)


# speedup vs baseline: 1.1287x; 1.0041x over previous
import functools

import jax
import jax.numpy as jnp
from jax import lax
from jax.experimental import pallas as pl
from jax.experimental.pallas import tpu as pltpu
from jax.experimental.pallas import tpu_sc as plsc

N = 10000
E = 320000
LATENT = 128
DOUT = 64

NC = 2
NS = 16
NW = NC * NS
BATCH = 64
NB = 160
CHB = 8
NCH = NB // CHB
EPW = NB * BATCH
EPAD = NW * EPW - E
ACCN = N + 1
RPS = 624
RTAIL = N - RPS * NS
ZROWS = 104
def _make_spmm(d, nbuf):
    assert CHB % nbuf == 0
    g = nbuf - 1
    mesh = plsc.VectorSubcoreMesh(core_axis_name="c", subcore_axis_name="s")

    @functools.partial(
        pl.kernel,
        out_type=jax.ShapeDtypeStruct((NC, N, d), jnp.float32),
        mesh=mesh,
        scratch_types=[
            pltpu.VMEM((2, CHB, BATCH), jnp.int32),
            pltpu.VMEM((2, CHB, BATCH), jnp.int32),
            pltpu.VMEM((nbuf, BATCH, d), jnp.float32),
            pltpu.VMEM((ZROWS, d), jnp.float32),
            pltpu.VMEM_SHARED((ACCN, d), jnp.float32),
            pltpu.SemaphoreType.DMA,
        ] + [pltpu.SemaphoreType.DMA] * (2 * nbuf),
    )
    def spmm(support, src, dst, zeros, out,
             src_v, dst_v, rows_v, zt, acc, isem, *sems):
        gs = sems[:nbuf]
        ss = sems[nbuf:]
        cid = lax.axis_index("c")
        sid = lax.axis_index("s")
        wid = sid * NC + cid

        def ichunk(c, b):
            pltpu.async_copy(src.at[wid, pl.ds(c * CHB, CHB)], src_v.at[b],
                             isem)
            pltpu.async_copy(dst.at[wid, pl.ds(c * CHB, CHB)], dst_v.at[b],
                             isem)

        def ichunk_wait():
            pltpu.make_async_copy(src.at[0, pl.ds(0, CHB)], src_v.at[0],
                                  isem).wait()
            pltpu.make_async_copy(dst.at[0, pl.ds(0, CHB)], dst_v.at[0],
                                  isem).wait()

        def gissue(buf, row, slot):
            pltpu.async_copy(support.at[src_v.at[buf, row]], rows_v.at[slot],
                             gs[slot])

        def gwait(slot):
            pltpu.make_async_copy(support.at[src_v.at[0, 0]],
                                  rows_v.at[slot], gs[slot]).wait()

        def sissue(buf, row, slot):
            pltpu.async_copy(rows_v.at[slot], acc.at[dst_v.at[buf, row]],
                             ss[slot], add=True)

        def swait(slot):
            pltpu.make_async_copy(rows_v.at[slot], acc.at[dst_v.at[0, 0]],
                                  ss[slot]).wait()

        ichunk(0, 0)
        off = pl.multiple_of(sid * RPS, 8)
        pltpu.sync_copy(zeros, zt)
        for i in range(RPS // ZROWS):
            pltpu.sync_copy(
                zt, acc.at[pl.ds(pl.multiple_of(off + i * ZROWS, 8), ZROWS)])

        @pl.when(sid == NS - 1)
        def _():
            pltpu.sync_copy(zt.at[pl.ds(0, RTAIL)],
                            acc.at[pl.ds(RPS * NS, RTAIL)])

        plsc.subcore_barrier()
        ichunk_wait()

        for k in range(g):
            gissue(0, k, k)

        def chunk_steps(x, first, last):
            buf = lax.rem(x, 2) if not isinstance(x, int) else x % 2
            nbf = (lax.rem(x + 1, 2) if not isinstance(x, int)
                   else (x + 1) % 2)
            for k in range(CHB):
                slot = k % nbuf
                pslot = (k + g) % nbuf
                gwait(slot)
                sissue(buf, k, slot)
                if not (first and k == 0):
                    swait(pslot)
                if k == 0 and not last:
                    ichunk(x + 1, nbf)
                if k + g < CHB:
                    gissue(buf, k + g, pslot)
                elif not last:
                    gissue(nbf, k + g - CHB, pslot)
                if k == CHB - g - 1 and not last:
                    ichunk_wait()

        chunk_steps(0, first=True, last=False)

        def body(x, carry):
            chunk_steps(x, first=False, last=False)
            return carry

        lax.fori_loop(1, NCH - 1, body, 0)
        chunk_steps(NCH - 1, first=False, last=True)
        swait((NB - 1) % nbuf)

        plsc.subcore_barrier()
        pltpu.sync_copy(acc.at[pl.ds(off, RPS)], out.at[cid, pl.ds(off, RPS)])

        @pl.when(sid == NS - 1)
        def _():
            pltpu.sync_copy(acc.at[pl.ds(RPS * NS, RTAIL)],
                            out.at[cid, pl.ds(RPS * NS, RTAIL)])

    return spmm


_spmm128 = _make_spmm(LATENT, nbuf=4)

BM = 1000
BDI = 1000
BDJ = 1280


def _tanh_mm(x, w):
    din, dout = w.shape

    def body(x_ref, w_ref, o_ref):
        o_ref[...] = jnp.tanh(
            jnp.dot(x_ref[...], w_ref[...], preferred_element_type=jnp.float32))

    return pl.pallas_call(
        body,
        grid=(N // BM,),
        in_specs=[pl.BlockSpec((BM, din), lambda i: (i, 0)),
                  pl.BlockSpec((din, dout), lambda i: (0, 0))],
        out_specs=pl.BlockSpec((BM, dout), lambda i: (i, 0)),
        out_shape=jax.ShapeDtypeStruct((N, dout), jnp.float32),
    )(x, w)


def _tanh_mm_partials(p, w):
    din, dout = w.shape

    def body(p_ref, w_ref, o_ref):
        x = p_ref[0] + p_ref[1]
        o_ref[...] = jnp.tanh(
            jnp.dot(x, w_ref[...], preferred_element_type=jnp.float32))

    return pl.pallas_call(
        body,
        grid=(N // BM,),
        in_specs=[pl.BlockSpec((2, BM, din), lambda i: (0, i, 0)),
                  pl.BlockSpec((din, dout), lambda i: (0, 0))],
        out_specs=pl.BlockSpec((BM, dout), lambda i: (i, 0)),
        out_shape=jax.ShapeDtypeStruct((N, dout), jnp.float32),
    )(p, w)


def _decode(p):
    def body(a_ref, b_ref, x_ref, adj_ref):
        xi = a_ref[0] + a_ref[1]
        xj = b_ref[0] + b_ref[1]
        x_ref[...] = xi[:, :DOUT]
        logits = lax.dot_general(xi, xj, (((1,), (1,)), ((), ())),
                                 preferred_element_type=jnp.float32)
        adj_ref[...] = jax.nn.sigmoid(logits)

    return pl.pallas_call(
        body,
        grid=(N // BDI, (N + BDJ - 1) // BDJ),
        in_specs=[pl.BlockSpec((2, BDI, LATENT), lambda i, j: (0, i, 0)),
                  pl.BlockSpec((2, BDJ, LATENT), lambda i, j: (0, j, 0))],
        out_specs=[pl.BlockSpec((BDI, DOUT), lambda i, j: (i, 0)),
                   pl.BlockSpec((BDI, BDJ), lambda i, j: (i, j))],
        out_shape=[jax.ShapeDtypeStruct((N, DOUT), jnp.float32),
                   jax.ShapeDtypeStruct((N, N), jnp.float32)],
    )(p, p)


def kernel(z_x, adj_edge_index, W4, W5, W6):
    dst = jnp.concatenate(
        [adj_edge_index[0], jnp.full((EPAD,), N, jnp.int32)]).reshape(
            NW, NB, BATCH)
    src = jnp.concatenate(
        [adj_edge_index[1], jnp.zeros((EPAD,), jnp.int32)]).reshape(
            NW, NB, BATCH)
    z128 = jnp.zeros((ZROWS, LATENT), jnp.float32)
    w6p = jnp.pad(W6, ((0, 0), (0, LATENT - DOUT)))

    s = _tanh_mm(z_x, W4)
    p = _spmm128(s, src, dst, z128)
    s = _tanh_mm_partials(p, W5)
    p = _spmm128(s, src, dst, z128)
    s = _tanh_mm_partials(p, w6p)
    p = _spmm128(s, src, dst, z128)
    x_hat, adj_hat = _decode(p)
    return (x_hat, adj_hat)


# speedup vs baseline: 1.2452x; 1.1032x over previous
import functools

import jax
import jax.numpy as jnp
from jax import lax
from jax.experimental import pallas as pl
from jax.experimental.pallas import tpu as pltpu
from jax.experimental.pallas import tpu_sc as plsc

N = 10000
E = 320000
LATENT = 128
DOUT = 64

NC = 2
NS = 16
NW = NC * NS
BATCH = 32
NB = 320
CHB = 16
NCH = NB // CHB
EPW = NB * BATCH
EPAD = NW * EPW - E
ACCN = N + 1
RPS = 624
RTAIL = N - RPS * NS
ZROWS = 104
def _make_spmm(d, nbuf):
    assert CHB % nbuf == 0
    g = nbuf - 1
    mesh = plsc.VectorSubcoreMesh(core_axis_name="c", subcore_axis_name="s")

    @functools.partial(
        pl.kernel,
        out_type=jax.ShapeDtypeStruct((NC, N, d), jnp.float32),
        mesh=mesh,
        scratch_types=[
            pltpu.VMEM((2, CHB, BATCH), jnp.int32),
            pltpu.VMEM((2, CHB, BATCH), jnp.int32),
            pltpu.VMEM((nbuf, BATCH, d), jnp.float32),
            pltpu.VMEM((ZROWS, d), jnp.float32),
            pltpu.VMEM_SHARED((ACCN, d), jnp.float32),
            pltpu.SemaphoreType.DMA,
        ] + [pltpu.SemaphoreType.DMA] * (2 * nbuf),
    )
    def spmm(support, src, dst, zeros, out,
             src_v, dst_v, rows_v, zt, acc, isem, *sems):
        gs = sems[:nbuf]
        ss = sems[nbuf:]
        cid = lax.axis_index("c")
        sid = lax.axis_index("s")
        wid = sid * NC + cid

        def ichunk(c, b):
            pltpu.async_copy(src.at[wid, pl.ds(c * CHB, CHB)], src_v.at[b],
                             isem)
            pltpu.async_copy(dst.at[wid, pl.ds(c * CHB, CHB)], dst_v.at[b],
                             isem)

        def ichunk_wait():
            pltpu.make_async_copy(src.at[0, pl.ds(0, CHB)], src_v.at[0],
                                  isem).wait()
            pltpu.make_async_copy(dst.at[0, pl.ds(0, CHB)], dst_v.at[0],
                                  isem).wait()

        def gissue(buf, row, slot):
            pltpu.async_copy(support.at[src_v.at[buf, row]], rows_v.at[slot],
                             gs[slot])

        def gwait(slot):
            pltpu.make_async_copy(support.at[src_v.at[0, 0]],
                                  rows_v.at[slot], gs[slot]).wait()

        def sissue(buf, row, slot):
            pltpu.async_copy(rows_v.at[slot], acc.at[dst_v.at[buf, row]],
                             ss[slot], add=True)

        def swait(slot):
            pltpu.make_async_copy(rows_v.at[slot], acc.at[dst_v.at[0, 0]],
                                  ss[slot]).wait()

        ichunk(0, 0)
        off = pl.multiple_of(sid * RPS, 8)
        pltpu.sync_copy(zeros, zt)
        for i in range(RPS // ZROWS):
            pltpu.sync_copy(
                zt, acc.at[pl.ds(pl.multiple_of(off + i * ZROWS, 8), ZROWS)])

        @pl.when(sid == NS - 1)
        def _():
            pltpu.sync_copy(zt.at[pl.ds(0, RTAIL)],
                            acc.at[pl.ds(RPS * NS, RTAIL)])

        plsc.subcore_barrier()
        ichunk_wait()

        for k in range(g):
            gissue(0, k, k)

        def chunk_steps(x, first, last):
            buf = lax.rem(x, 2) if not isinstance(x, int) else x % 2
            nbf = (lax.rem(x + 1, 2) if not isinstance(x, int)
                   else (x + 1) % 2)
            for k in range(CHB):
                slot = k % nbuf
                pslot = (k + g) % nbuf
                gwait(slot)
                sissue(buf, k, slot)
                if not (first and k == 0):
                    swait(pslot)
                if k == 0 and not last:
                    ichunk(x + 1, nbf)
                if k + g < CHB:
                    gissue(buf, k + g, pslot)
                elif not last:
                    gissue(nbf, k + g - CHB, pslot)
                if k == CHB - g - 1 and not last:
                    ichunk_wait()

        chunk_steps(0, first=True, last=False)

        def body(x, carry):
            chunk_steps(x, first=False, last=False)
            return carry

        lax.fori_loop(1, NCH - 1, body, 0)
        chunk_steps(NCH - 1, first=False, last=True)
        swait((NB - 1) % nbuf)

        plsc.subcore_barrier()
        pltpu.sync_copy(acc.at[pl.ds(off, RPS)], out.at[cid, pl.ds(off, RPS)])

        @pl.when(sid == NS - 1)
        def _():
            pltpu.sync_copy(acc.at[pl.ds(RPS * NS, RTAIL)],
                            out.at[cid, pl.ds(RPS * NS, RTAIL)])

    return spmm


_spmm128 = _make_spmm(LATENT, nbuf=4)

BM = 1000
BDI = 1000
BDJ = 1280


def _tanh_mm(x, w):
    din, dout = w.shape

    def body(x_ref, w_ref, o_ref):
        o_ref[...] = jnp.tanh(
            jnp.dot(x_ref[...], w_ref[...], preferred_element_type=jnp.float32))

    return pl.pallas_call(
        body,
        grid=(N // BM,),
        in_specs=[pl.BlockSpec((BM, din), lambda i: (i, 0)),
                  pl.BlockSpec((din, dout), lambda i: (0, 0))],
        out_specs=pl.BlockSpec((BM, dout), lambda i: (i, 0)),
        out_shape=jax.ShapeDtypeStruct((N, dout), jnp.float32),
    )(x, w)


def _tanh_mm_partials(p, w):
    din, dout = w.shape

    def body(p_ref, w_ref, o_ref):
        x = p_ref[0] + p_ref[1]
        o_ref[...] = jnp.tanh(
            jnp.dot(x, w_ref[...], preferred_element_type=jnp.float32))

    return pl.pallas_call(
        body,
        grid=(N // BM,),
        in_specs=[pl.BlockSpec((2, BM, din), lambda i: (0, i, 0)),
                  pl.BlockSpec((din, dout), lambda i: (0, 0))],
        out_specs=pl.BlockSpec((BM, dout), lambda i: (i, 0)),
        out_shape=jax.ShapeDtypeStruct((N, dout), jnp.float32),
    )(p, w)


def _decode(p):
    def body(a_ref, b_ref, x_ref, adj_ref):
        xi = a_ref[0] + a_ref[1]
        xj = b_ref[0] + b_ref[1]
        x_ref[...] = xi[:, :DOUT]
        logits = lax.dot_general(xi, xj, (((1,), (1,)), ((), ())),
                                 preferred_element_type=jnp.float32)
        adj_ref[...] = jax.nn.sigmoid(logits)

    return pl.pallas_call(
        body,
        grid=(N // BDI, (N + BDJ - 1) // BDJ),
        in_specs=[pl.BlockSpec((2, BDI, LATENT), lambda i, j: (0, i, 0)),
                  pl.BlockSpec((2, BDJ, LATENT), lambda i, j: (0, j, 0))],
        out_specs=[pl.BlockSpec((BDI, DOUT), lambda i, j: (i, 0)),
                   pl.BlockSpec((BDI, BDJ), lambda i, j: (i, j))],
        out_shape=[jax.ShapeDtypeStruct((N, DOUT), jnp.float32),
                   jax.ShapeDtypeStruct((N, N), jnp.float32)],
    )(p, p)


def kernel(z_x, adj_edge_index, W4, W5, W6):
    dst = jnp.concatenate(
        [adj_edge_index[0], jnp.full((EPAD,), N, jnp.int32)]).reshape(
            NW, NB, BATCH)
    src = jnp.concatenate(
        [adj_edge_index[1], jnp.zeros((EPAD,), jnp.int32)]).reshape(
            NW, NB, BATCH)
    z128 = jnp.zeros((ZROWS, LATENT), jnp.float32)
    w6p = jnp.pad(W6, ((0, 0), (0, LATENT - DOUT)))

    s = _tanh_mm(z_x, W4)
    p = _spmm128(s, src, dst, z128)
    s = _tanh_mm_partials(p, W5)
    p = _spmm128(s, src, dst, z128)
    s = _tanh_mm_partials(p, w6p)
    p = _spmm128(s, src, dst, z128)
    x_hat, adj_hat = _decode(p)
    return (x_hat, adj_hat)
